# baseline (speedup 1.0000x reference)
import functools

import jax
import jax.numpy as jnp
from jax import lax
from jax.experimental import pallas as pl
from jax.experimental.pallas import tpu as pltpu
from jax.experimental.pallas import tpu_sc as plsc

_EPS = 1e-12
_TILE = 256
_CHUNK = 128


def _entity_gather_sc(table, ids_flat):
    info = plsc.get_sparse_core_info()
    num_cores = info.num_cores
    nw = num_cores * info.num_subcores
    n = ids_flat.shape[0]
    d = table.shape[1]
    n_per_w = n // nw
    assert n_per_w * nw == n and n_per_w % _CHUNK == 0
    n_ch = n_per_w // _CHUNK
    mesh = plsc.VectorSubcoreMesh(core_axis_name="c", subcore_axis_name="s")

    @functools.partial(
        pl.kernel,
        mesh=mesh,
        compiler_params=pltpu.CompilerParams(use_tc_tiling_on_sc=True),
        out_type=jax.ShapeDtypeStruct((n, d), jnp.float32),
        scratch_types=[
            pltpu.VMEM((n_per_w,), jnp.int32),
            pltpu.VMEM((2, _CHUNK, d), jnp.float32),
            pltpu.SemaphoreType.DMA,
            pltpu.SemaphoreType.DMA,
            pltpu.SemaphoreType.DMA,
        ],
    )
    def gather_kernel(table_hbm, idx_hbm, out_hbm, idx_v, rows_v, gsem, ssem0,
                      ssem1):
        wid = lax.axis_index("s") * num_cores + lax.axis_index("c")
        base = wid * n_per_w
        pltpu.sync_copy(idx_hbm.at[pl.ds(base, n_per_w)], idx_v)
        ssems = (ssem0, ssem1)
        scatters = [None, None]
        for c in range(n_ch):
            buf = c % 2
            if scatters[buf] is not None:
                scatters[buf].wait()
            pltpu.async_copy(
                table_hbm.at[idx_v.at[pl.ds(c * _CHUNK, _CHUNK)]],
                rows_v.at[buf],
                gsem,
            ).wait()
            scatters[buf] = pltpu.async_copy(
                rows_v.at[buf],
                out_hbm.at[pl.ds(base + c * _CHUNK, _CHUNK)],
                ssems[buf],
            )
        for s in scatters:
            if s is not None:
                s.wait()

    return gather_kernel(table, ids_flat)


def _tc_fused(pos_ids, gathered, dense_w, pos_table, gamma, beta):
    n, m = pos_ids.shape
    emb = gathered.shape[1]
    maxpos, hid = pos_table.shape
    grid = (n // _TILE,)

    def body(pos_ids_ref, ent_ref, dense_w_ref, pos_table_ref, g_ref, b_ref,
             out_ref):
        ids = pos_ids_ref[...]
        pos_iota = lax.broadcasted_iota(jnp.int32, (_TILE, maxpos), 1)
        oh = (ids[:, 0][:, None] == pos_iota).astype(jnp.float32)
        for j in range(1, m):
            oh += (ids[:, j][:, None] == pos_iota).astype(jnp.float32)
        cnt = jnp.sum((ids != 0).astype(jnp.float32), axis=1, keepdims=True)
        denom = jnp.maximum(cnt, 1.0)
        pos_sum = jnp.dot(oh, pos_table_ref[...],
                          preferred_element_type=jnp.float32)
        ent = jnp.dot(ent_ref[...], dense_w_ref[...],
                      preferred_element_type=jnp.float32)
        x = ent + pos_sum / denom
        mu = jnp.mean(x, axis=-1, keepdims=True)
        xc = x - mu
        var = jnp.mean(xc * xc, axis=-1, keepdims=True)
        y = xc * lax.rsqrt(var + _EPS)
        out_ref[...] = y * g_ref[...] + b_ref[...]

    return pl.pallas_call(
        body,
        grid=grid,
        in_specs=[
            pl.BlockSpec((_TILE, m), lambda i: (i, 0)),
            pl.BlockSpec((_TILE, emb), lambda i: (i, 0)),
            pl.BlockSpec((emb, hid), lambda i: (0, 0)),
            pl.BlockSpec((maxpos, hid), lambda i: (0, 0)),
            pl.BlockSpec((1, hid), lambda i: (0, 0)),
            pl.BlockSpec((1, hid), lambda i: (0, 0)),
        ],
        out_specs=pl.BlockSpec((_TILE, hid), lambda i: (i, 0)),
        out_shape=jax.ShapeDtypeStruct((n, hid), jnp.float32),
    )(pos_ids, gathered, dense_w, pos_table, gamma, beta)


def kernel(entity_ids, entity_position_ids, entity_table, pos_table, dense_w,
           ln_gamma, ln_beta):
    b, l = entity_ids.shape
    m = entity_position_ids.shape[-1]
    hid = pos_table.shape[1]
    n = b * l
    ids_flat = entity_ids.reshape(n).astype(jnp.int32)
    gathered = _entity_gather_sc(entity_table, ids_flat)
    pos_flat = entity_position_ids.reshape(n, m).astype(jnp.int32)
    out = _tc_fused(pos_flat, gathered, dense_w, pos_table,
                    ln_gamma.reshape(1, hid), ln_beta.reshape(1, hid))
    return out.reshape(b, l, hid)


# speedup vs baseline: 1.3485x; 1.3485x over previous
import functools

import jax
import jax.numpy as jnp
from jax import lax
from jax.experimental import pallas as pl
from jax.experimental.pallas import tpu as pltpu
from jax.experimental.pallas import tpu_sc as plsc

_EPS = 1e-12
_TILE_B = 16
_CHUNK = 128


def _entity_gather_sc(table, ids_flat):
    info = plsc.get_sparse_core_info()
    num_cores = info.num_cores
    nw = num_cores * info.num_subcores
    n = ids_flat.shape[0]
    d = table.shape[1]
    n_per_w = n // nw
    assert n_per_w * nw == n and n_per_w % _CHUNK == 0
    n_ch = n_per_w // _CHUNK
    mesh = plsc.VectorSubcoreMesh(core_axis_name="c", subcore_axis_name="s")

    @functools.partial(
        pl.kernel,
        mesh=mesh,
        compiler_params=pltpu.CompilerParams(use_tc_tiling_on_sc=True),
        out_type=jax.ShapeDtypeStruct((n, d), jnp.float32),
        scratch_types=[
            pltpu.VMEM((n_per_w,), jnp.int32),
            pltpu.VMEM((2, _CHUNK, d), jnp.float32),
            pltpu.SemaphoreType.DMA,
            pltpu.SemaphoreType.DMA,
            pltpu.SemaphoreType.DMA,
        ],
    )
    def gather_kernel(table_hbm, idx_hbm, out_hbm, idx_v, rows_v, gsem, ssem0,
                      ssem1):
        wid = lax.axis_index("s") * num_cores + lax.axis_index("c")
        base = wid * n_per_w
        pltpu.sync_copy(idx_hbm.at[pl.ds(base, n_per_w)], idx_v)
        ssems = (ssem0, ssem1)
        scatters = [None, None]
        for c in range(n_ch):
            buf = c % 2
            if scatters[buf] is not None:
                scatters[buf].wait()
            pltpu.async_copy(
                table_hbm.at[idx_v.at[pl.ds(c * _CHUNK, _CHUNK)]],
                rows_v.at[buf],
                gsem,
            ).wait()
            scatters[buf] = pltpu.async_copy(
                rows_v.at[buf],
                out_hbm.at[pl.ds(base + c * _CHUNK, _CHUNK)],
                ssems[buf],
            )
        for s in scatters:
            if s is not None:
                s.wait()

    return gather_kernel(table, ids_flat)


def _tc_fused(pos_ids3, gathered, dense_w, pos_table, gamma, beta):
    b, l, m = pos_ids3.shape
    emb = gathered.shape[1]
    maxpos, hid = pos_table.shape
    tb = _TILE_B
    rows = tb * l
    grid = (b // tb,)

    def body(pos_ids_ref, ent_ref, dense_w_ref, pos_table_ref, g_ref, b_ref,
             out_ref):
        ids = pos_ids_ref[...].reshape(rows, m)
        pos_iota = lax.broadcasted_iota(jnp.int32, (rows, maxpos), 1)
        oh = (ids[:, 0][:, None] == pos_iota).astype(jnp.float32)
        for j in range(1, m):
            oh += (ids[:, j][:, None] == pos_iota).astype(jnp.float32)
        cnt = jnp.sum((ids != 0).astype(jnp.float32), axis=1, keepdims=True)
        denom = jnp.maximum(cnt, 1.0)
        pos_sum = jnp.dot(oh, pos_table_ref[...],
                          preferred_element_type=jnp.float32)
        ent = jnp.dot(ent_ref[...], dense_w_ref[...],
                      preferred_element_type=jnp.float32)
        x = ent + pos_sum / denom
        mu = jnp.mean(x, axis=-1, keepdims=True)
        xc = x - mu
        var = jnp.mean(xc * xc, axis=-1, keepdims=True)
        y = xc * lax.rsqrt(var + _EPS)
        out_ref[...] = (y * g_ref[...] + b_ref[...]).reshape(tb, l, hid)

    return pl.pallas_call(
        body,
        grid=grid,
        in_specs=[
            pl.BlockSpec((tb, l, m), lambda i: (i, 0, 0)),
            pl.BlockSpec((rows, emb), lambda i: (i, 0)),
            pl.BlockSpec((emb, hid), lambda i: (0, 0)),
            pl.BlockSpec((maxpos, hid), lambda i: (0, 0)),
            pl.BlockSpec((1, hid), lambda i: (0, 0)),
            pl.BlockSpec((1, hid), lambda i: (0, 0)),
        ],
        out_specs=pl.BlockSpec((tb, l, hid), lambda i: (i, 0, 0)),
        out_shape=jax.ShapeDtypeStruct((b, l, hid), jnp.float32),
    )(pos_ids3, gathered, dense_w, pos_table, gamma, beta)


def kernel(entity_ids, entity_position_ids, entity_table, pos_table, dense_w,
           ln_gamma, ln_beta):
    b, l = entity_ids.shape
    hid = pos_table.shape[1]
    n = b * l
    ids_flat = entity_ids.reshape(n)
    gathered = _entity_gather_sc(entity_table, ids_flat)
    return _tc_fused(entity_position_ids, gathered, dense_w, pos_table,
                     ln_gamma.reshape(1, hid), ln_beta.reshape(1, hid))
